# baseline (speedup 1.0000x reference)
import jax
import jax.numpy as jnp
from jax import lax
from jax.experimental import pallas as pl
from jax.experimental.pallas import tpu as pltpu
from jax.experimental.pallas import tpu_sc as plsc

PI = 3.1415926235897933
NUM_ENTITY = 100000
NUM_RELATION = 1000
NUM_TIME = 365
NUM_MAXLEN = 50
DIM = 128
BATCH = 16384

NC = 2
NS = 16
NW = NC * NS
B2 = 2 * BATCH
PW = B2 // NW
C = 32
NCHUNK = PW // C
NBUF = 4
assert NCHUNK % NBUF == 0
G = DIM // 16

_MAGIC = 12582912.0
_Q1 = 1.57079101
_Q3 = -0.64589288
_Q5 = 0.07943441
_Q7 = -0.00433314

_GDN = lax.GatherDimensionNumbers(
    offset_dims=(), collapsed_slice_dims=(0,), start_index_map=(0,))


def _abs_sin_q(v):
    n = (v * 0.25 + _MAGIC) - _MAGIC
    r = v - n * 4.0
    a = jnp.abs(r)
    m = jnp.minimum(a, 2.0 - a)
    u = m * m
    p = ((_Q7 * u + _Q5) * u + _Q3) * u + _Q1
    return m * p


def _lane_sum(x, bfly):
    for idx in bfly:
        x = x + lax.gather(x, idx, _GDN, slice_sizes=(1,),
                           mode=lax.GatherScatterMode.PROMISE_IN_BOUNDS)
    return x


def _sqrt(x):
    bits = lax.bitcast_convert_type(x, jnp.int32)
    y = lax.bitcast_convert_type(
        0x5F3759DF - lax.shift_right_logical(bits, 1), jnp.float32)
    for _ in range(3):
        y = y * (1.5 - 0.5 * x * y * y)
    return x * y


def _unpack_i32(v):
    va = lax.bitcast_convert_type(lax.shift_left(v, 16), jnp.float32)
    vb = lax.bitcast_convert_type(
        lax.bitwise_and(v, jnp.int32(-65536)), jnp.float32)
    return va, vb


def _sc_body(h_hbm, r_hbm, t_hbm, tm_hbm, st_hbm,
             ent_hbm, relmod_hbm, mode_hbm, time_hbm, step_hbm,
             out_hbm,
             idx_h, idx_r, idx_t, idx_tm, idx_st,
             step_tab, time_tab,
             hbuf, tbuf, hmbuf, rmbuf,
             res_p, *sems):
    wid = lax.axis_index("s") * NC + lax.axis_index("c")
    lane = lax.iota(jnp.int32, 16)
    bfly = [lax.bitwise_and(lane + sh, 15)[:, None] for sh in (8, 4, 2, 1)]

    pltpu.sync_copy(h_hbm.at[wid], idx_h)
    pltpu.sync_copy(r_hbm.at[wid], idx_r)
    pltpu.sync_copy(t_hbm.at[wid], idx_t)
    pltpu.sync_copy(tm_hbm.at[wid], idx_tm.at[pl.ds(0, PW)])
    pltpu.sync_copy(st_hbm.at[wid], idx_st.at[pl.ds(0, PW)])
    pltpu.sync_copy(step_hbm, step_tab)
    pltpu.sync_copy(time_hbm, time_tab)

    for g in range(G):
        sl = pl.ds(g * 16, 16)
        step_tab[0, sl] = jnp.abs(step_tab[0, sl])

    def cum_body(i, carry):
        for g in range(G):
            sl = pl.ds(g * 16, 16)
            step_tab[i, sl] = jnp.abs(step_tab[i, sl]) + step_tab[i - 1, sl]
        return carry

    lax.fori_loop(1, NUM_MAXLEN, cum_body, 0)

    def chunk_copies(c, b):
        return [
            pltpu.make_async_copy(ent_hbm.at[idx_h.at[c]], hbuf.at[b], sems[b]),
            pltpu.make_async_copy(ent_hbm.at[idx_t.at[c]], tbuf.at[b], sems[b]),
            pltpu.make_async_copy(mode_hbm.at[idx_h.at[c]], hmbuf.at[b], sems[b]),
            pltpu.make_async_copy(relmod_hbm.at[idx_r.at[c]], rmbuf.at[b],
                                  sems[b]),
        ]

    for b in range(NBUF):
        for cp in chunk_copies(b, b):
            cp.start()

    def compute_chunk(c, b):
        hb, tb, hmb, rmb = (hbuf.at[b], tbuf.at[b], hmbuf.at[b],
                            rmbuf.at[b])

        def grp_body(q, carry2):
            def elem_body(l, carry3):
                tp, tq = carry3
                j = q * 16 + l
                e = c * C + j
                tm_i = idx_tm[pl.ds(e, 16)][0]
                st_i = idx_st[pl.ds(e, 16)][0]
                accp = jnp.zeros((16,), jnp.float32)
                accm = jnp.zeros((16,), jnp.float32)
                tbase = tm_i * (DIM // 2)
                for gp in range(G // 2):
                    sl16 = pl.ds(gp * 16, 16)
                    r0, r1 = _unpack_i32(rmb[j, sl16])
                    rm0, rm1 = _unpack_i32(
                        rmb[j, pl.ds(DIM // 2 + gp * 16, 16)])
                    tm0, tm1 = _unpack_i32(
                        time_tab[pl.ds(tbase + gp * 16, 16)])
                    for k, (rr, rmv, tmv) in enumerate(
                            ((r0, rm0, tm0), (r1, rm1, tm1))):
                        sl = pl.ds((gp * 2 + k) * 16, 16)
                        h = hb[j, sl]
                        t = tb[j, sl]
                        stv = step_tab[st_i, sl]
                        w = (h - t) * (tmv + 1.0) + rr
                        accp = accp + _abs_sin_q(w)
                        dm = jnp.abs(hmb[j, sl]) * jnp.abs(rmv) - stv
                        accm = accm + dm * dm
                sel = lane == l
                tp = jnp.where(sel, _lane_sum(accp, bfly), tp)
                tq = jnp.where(sel, _lane_sum(accm, bfly), tq)
                return (tp, tq)

            z = jnp.zeros((16,), jnp.float32)
            tp, tq = lax.fori_loop(0, 16, elem_body, (z, z))
            base16 = c * C + q * 16
            res_p[pl.ds(base16, 16)] = tp + 0.5 * _sqrt(tq)
            return carry2

        lax.fori_loop(0, C // 16, grp_body, 0)

    def pipe_body(p, carry):
        for b in range(NBUF):
            c = NBUF * p + b
            for cp in chunk_copies(c, b):
                cp.wait()
            compute_chunk(c, b)
            nxt = jnp.minimum(c + NBUF, NCHUNK - 1)
            for cp in chunk_copies(nxt, b):
                cp.start()
        return carry

    lax.fori_loop(0, NCHUNK // NBUF, pipe_body, 0)
    for b in range(NBUF):
        for cp in chunk_copies(NCHUNK - 1, b):
            cp.wait()
    pltpu.sync_copy(res_p, out_hbm.at[wid])


@jax.jit
def _mpke_sc(h, r, t, tm, st, ent, relmod, mode, time_pk, stepw):
    mesh = plsc.VectorSubcoreMesh(core_axis_name="c", subcore_axis_name="s")
    f32 = jnp.float32
    i32 = jnp.int32
    run = pl.kernel(
        _sc_body,
        out_type=jax.ShapeDtypeStruct((NW, PW), f32),
        mesh=mesh,
        scratch_types=[
            pltpu.VMEM((NCHUNK, C), i32),
            pltpu.VMEM((NCHUNK, C), i32),
            pltpu.VMEM((NCHUNK, C), i32),
            pltpu.VMEM((PW + 16,), i32),
            pltpu.VMEM((PW + 16,), i32),
            pltpu.VMEM((NUM_MAXLEN, DIM), f32),
            pltpu.VMEM((NUM_TIME * DIM // 2,), i32),
            pltpu.VMEM((NBUF, C, DIM), f32),
            pltpu.VMEM((NBUF, C, DIM), f32),
            pltpu.VMEM((NBUF, C, DIM), f32),
            pltpu.VMEM((NBUF, C, DIM), i32),
            pltpu.VMEM((PW,), f32),
        ] + [pltpu.SemaphoreType.DMA] * NBUF,
    )
    return run(h, r, t, tm, st, ent, relmod, mode, time_pk, stepw)


def _pack_half(tab):
    rows = tab.shape[0]
    t = tab.reshape(rows, DIM // 32, 2, 16).transpose(0, 1, 3, 2)
    tb = t.astype(jnp.bfloat16)
    return lax.bitcast_convert_type(tb, jnp.int32).reshape(rows, DIM // 2)


def kernel(positiveBatchHead, positiveBatchRelation, positiveBatchTail,
           positiveBatchTime, positiveBatchStep,
           corruptedBatchHead, corruptedBatchRelation, corruptedBatchTail,
           corruptedBatchTime, corruptedBatchStep,
           entity_weight, relation_weight, mod_e_weight, mod_r_weight,
           time_weight, step_weight):
    def prep(a, b):
        return (jnp.concatenate([a, b]).astype(jnp.int32)
                .reshape(NW, NCHUNK, C))

    h = prep(positiveBatchHead, corruptedBatchHead)
    r = prep(positiveBatchRelation, corruptedBatchRelation)
    t = prep(positiveBatchTail, corruptedBatchTail)
    tm = prep(positiveBatchTime, corruptedBatchTime).reshape(NW, PW)
    st = prep(positiveBatchStep, corruptedBatchStep).reshape(NW, PW)
    relmod = jnp.concatenate(
        [_pack_half(relation_weight), _pack_half(mod_r_weight)], axis=1)
    out = _mpke_sc(h, r, t, tm, st, entity_weight, relmod, mod_e_weight,
                   _pack_half(time_weight).reshape(NUM_TIME * DIM // 2),
                   step_weight)
    loss = out.reshape(B2)
    return (loss[:BATCH], loss[BATCH:])


# speedup vs baseline: 1.0587x; 1.0587x over previous
import jax
import jax.numpy as jnp
from jax import lax
from jax.experimental import pallas as pl
from jax.experimental.pallas import tpu as pltpu
from jax.experimental.pallas import tpu_sc as plsc

PI = 3.1415926235897933
NUM_ENTITY = 100000
NUM_RELATION = 1000
NUM_TIME = 365
NUM_MAXLEN = 50
DIM = 128
BATCH = 16384

NC = 2
NS = 16
NW = NC * NS
B2 = 2 * BATCH
PW = B2 // NW
C = 32
NCHUNK = PW // C
NBUF = 4
assert NCHUNK % NBUF == 0
G = DIM // 16

_MAGIC = 12582912.0
_Q1 = 1.57079101
_Q3 = -0.64589288
_Q5 = 0.07943441
_Q7 = -0.00433314

_GDN = lax.GatherDimensionNumbers(
    offset_dims=(), collapsed_slice_dims=(0,), start_index_map=(0,))


def _abs_sin_q(v):
    n = (v * 0.25 + _MAGIC) - _MAGIC
    r = v - n * 4.0
    a = jnp.abs(r)
    m = jnp.minimum(a, 2.0 - a)
    u = m * m
    p = ((_Q7 * u + _Q5) * u + _Q3) * u + _Q1
    return m * p


def _lane_sum(x, bfly):
    for idx in bfly:
        x = x + lax.gather(x, idx, _GDN, slice_sizes=(1,),
                           mode=lax.GatherScatterMode.PROMISE_IN_BOUNDS)
    return x


def _sqrt(x):
    bits = lax.bitcast_convert_type(x, jnp.int32)
    y = lax.bitcast_convert_type(
        0x5F3759DF - lax.shift_right_logical(bits, 1), jnp.float32)
    for _ in range(3):
        y = y * (1.5 - 0.5 * x * y * y)
    return x * y


def _unpack_i32(v):
    va = lax.bitcast_convert_type(lax.shift_left(v, 16), jnp.float32)
    vb = lax.bitcast_convert_type(
        lax.bitwise_and(v, jnp.int32(-65536)), jnp.float32)
    return va, vb


def _sc_body(h_hbm, r_hbm, t_hbm, tm_hbm, st_hbm,
             ent_hbm, rel_hbm, modr_hbm, mode_hbm, time_hbm, step_hbm,
             out_hbm,
             idx_h, idx_r, idx_t, idx_tm, idx_st,
             step_tab, time_tab,
             hbuf, tbuf, hmbuf, rbuf, rmbuf,
             res_p, *sems):
    wid = lax.axis_index("s") * NC + lax.axis_index("c")
    lane = lax.iota(jnp.int32, 16)
    bfly = [lax.bitwise_and(lane + sh, 15)[:, None] for sh in (8, 4, 2, 1)]

    pltpu.sync_copy(h_hbm.at[wid], idx_h)
    pltpu.sync_copy(r_hbm.at[wid], idx_r)
    pltpu.sync_copy(t_hbm.at[wid], idx_t)
    pltpu.sync_copy(tm_hbm.at[wid], idx_tm.at[pl.ds(0, PW)])
    pltpu.sync_copy(st_hbm.at[wid], idx_st.at[pl.ds(0, PW)])
    pltpu.sync_copy(step_hbm, step_tab)
    pltpu.sync_copy(time_hbm, time_tab)

    for g in range(G):
        sl = pl.ds(g * 16, 16)
        step_tab[0, sl] = jnp.abs(step_tab[0, sl])

    def cum_body(i, carry):
        for g in range(G):
            sl = pl.ds(g * 16, 16)
            step_tab[i, sl] = jnp.abs(step_tab[i, sl]) + step_tab[i - 1, sl]
        return carry

    lax.fori_loop(1, NUM_MAXLEN, cum_body, 0)

    def chunk_copies(c, b):
        return [
            pltpu.make_async_copy(ent_hbm.at[idx_h.at[c]], hbuf.at[b], sems[b]),
            pltpu.make_async_copy(ent_hbm.at[idx_t.at[c]], tbuf.at[b], sems[b]),
            pltpu.make_async_copy(mode_hbm.at[idx_h.at[c]], hmbuf.at[b], sems[b]),
            pltpu.make_async_copy(rel_hbm.at[idx_r.at[c]], rbuf.at[b], sems[b]),
            pltpu.make_async_copy(modr_hbm.at[idx_r.at[c]], rmbuf.at[b],
                                  sems[b]),
        ]

    for b in range(NBUF):
        for cp in chunk_copies(b, b):
            cp.start()

    def compute_chunk(c, b):
        hb, tb, hmb, rb, rmb = (hbuf.at[b], tbuf.at[b], hmbuf.at[b],
                                rbuf.at[b], rmbuf.at[b])

        def grp_body(q, carry2):
            def elem_body(l, carry3):
                tp, tq = carry3
                j = q * 16 + l
                e = c * C + j
                tm_i = idx_tm[pl.ds(e, 16)][0]
                st_i = idx_st[pl.ds(e, 16)][0]
                accp = jnp.zeros((16,), jnp.float32)
                accm = jnp.zeros((16,), jnp.float32)
                tbase = tm_i * (DIM // 2)
                for gp in range(G // 2):
                    tm0, tm1 = _unpack_i32(
                        time_tab[pl.ds(tbase + gp * 16, 16)])
                    for k, tmv in enumerate((tm0, tm1)):
                        sl = pl.ds((gp * 2 + k) * 16, 16)
                        h = hb[j, sl]
                        t = tb[j, sl]
                        stv = step_tab[st_i, sl]
                        w = (h - t) * (tmv + 1.0) + rb[j, sl]
                        accp = accp + _abs_sin_q(w)
                        dm = jnp.abs(hmb[j, sl]) * jnp.abs(rmb[j, sl]) - stv
                        accm = accm + dm * dm
                sel = lane == l
                tp = jnp.where(sel, _lane_sum(accp, bfly), tp)
                tq = jnp.where(sel, _lane_sum(accm, bfly), tq)
                return (tp, tq)

            z = jnp.zeros((16,), jnp.float32)
            tp, tq = lax.fori_loop(0, 16, elem_body, (z, z))
            base16 = c * C + q * 16
            res_p[pl.ds(base16, 16)] = tp + 0.5 * _sqrt(tq)
            return carry2

        lax.fori_loop(0, C // 16, grp_body, 0)

    def pipe_body(p, carry):
        for b in range(NBUF):
            c = NBUF * p + b
            for cp in chunk_copies(c, b):
                cp.wait()
            compute_chunk(c, b)
            nxt = jnp.minimum(c + NBUF, NCHUNK - 1)
            for cp in chunk_copies(nxt, b):
                cp.start()
        return carry

    lax.fori_loop(0, NCHUNK // NBUF, pipe_body, 0)
    for b in range(NBUF):
        for cp in chunk_copies(NCHUNK - 1, b):
            cp.wait()
    pltpu.sync_copy(res_p, out_hbm.at[wid])


@jax.jit
def _mpke_sc(h, r, t, tm, st, ent, rel, modr, mode, time_pk, stepw):
    mesh = plsc.VectorSubcoreMesh(core_axis_name="c", subcore_axis_name="s")
    f32 = jnp.float32
    i32 = jnp.int32
    run = pl.kernel(
        _sc_body,
        out_type=jax.ShapeDtypeStruct((NW, PW), f32),
        mesh=mesh,
        scratch_types=[
            pltpu.VMEM((NCHUNK, C), i32),
            pltpu.VMEM((NCHUNK, C), i32),
            pltpu.VMEM((NCHUNK, C), i32),
            pltpu.VMEM((PW + 16,), i32),
            pltpu.VMEM((PW + 16,), i32),
            pltpu.VMEM((NUM_MAXLEN, DIM), f32),
            pltpu.VMEM((NUM_TIME * DIM // 2,), i32),
            pltpu.VMEM((NBUF, C, DIM), f32),
            pltpu.VMEM((NBUF, C, DIM), f32),
            pltpu.VMEM((NBUF, C, DIM), f32),
            pltpu.VMEM((NBUF, C, DIM), f32),
            pltpu.VMEM((NBUF, C, DIM), f32),
            pltpu.VMEM((PW,), f32),
        ] + [pltpu.SemaphoreType.DMA] * NBUF,
    )
    return run(h, r, t, tm, st, ent, rel, modr, mode, time_pk, stepw)


def _pack_half(tab):
    rows = tab.shape[0]
    t = tab.reshape(rows, DIM // 32, 2, 16).transpose(0, 1, 3, 2)
    tb = t.astype(jnp.bfloat16)
    return lax.bitcast_convert_type(tb, jnp.int32).reshape(rows, DIM // 2)


def kernel(positiveBatchHead, positiveBatchRelation, positiveBatchTail,
           positiveBatchTime, positiveBatchStep,
           corruptedBatchHead, corruptedBatchRelation, corruptedBatchTail,
           corruptedBatchTime, corruptedBatchStep,
           entity_weight, relation_weight, mod_e_weight, mod_r_weight,
           time_weight, step_weight):
    def prep(a, b):
        return (jnp.concatenate([a, b]).astype(jnp.int32)
                .reshape(NW, NCHUNK, C))

    h = prep(positiveBatchHead, corruptedBatchHead)
    r = prep(positiveBatchRelation, corruptedBatchRelation)
    t = prep(positiveBatchTail, corruptedBatchTail)
    tm = prep(positiveBatchTime, corruptedBatchTime).reshape(NW, PW)
    st = prep(positiveBatchStep, corruptedBatchStep).reshape(NW, PW)
    out = _mpke_sc(h, r, t, tm, st, entity_weight, relation_weight,
                   mod_r_weight, mod_e_weight,
                   _pack_half(time_weight).reshape(NUM_TIME * DIM // 2),
                   step_weight)
    loss = out.reshape(B2)
    return (loss[:BATCH], loss[BATCH:])


# speedup vs baseline: 1.1866x; 1.1208x over previous
import jax
import jax.numpy as jnp
from jax import lax
from jax.experimental import pallas as pl
from jax.experimental.pallas import tpu as pltpu
from jax.experimental.pallas import tpu_sc as plsc

PI = 3.1415926235897933
NUM_ENTITY = 100000
NUM_RELATION = 1000
NUM_TIME = 365
NUM_MAXLEN = 50
DIM = 128
BATCH = 16384

NC = 2
NS = 16
NW = NC * NS
B2 = 2 * BATCH
PW = B2 // NW
C = 32
NCHUNK = PW // C
NBUF = 2
assert NCHUNK % NBUF == 0
G = DIM // 16

_MAGIC = 12582912.0
_Q1 = 1.57079101
_Q3 = -0.64589288
_Q5 = 0.07943441
_Q7 = -0.00433314

_GDN = lax.GatherDimensionNumbers(
    offset_dims=(), collapsed_slice_dims=(0,), start_index_map=(0,))


def _abs_sin_q(v):
    n = (v * 0.25 + _MAGIC) - _MAGIC
    r = v - n * 4.0
    a = jnp.abs(r)
    m = jnp.minimum(a, 2.0 - a)
    u = m * m
    p = ((_Q7 * u + _Q5) * u + _Q3) * u + _Q1
    return m * p


def _lane_sum(x, bfly):
    for idx in bfly:
        x = x + lax.gather(x, idx, _GDN, slice_sizes=(1,),
                           mode=lax.GatherScatterMode.PROMISE_IN_BOUNDS)
    return x


def _sqrt(x):
    bits = lax.bitcast_convert_type(x, jnp.int32)
    y = lax.bitcast_convert_type(
        0x5F3759DF - lax.shift_right_logical(bits, 1), jnp.float32)
    for _ in range(3):
        y = y * (1.5 - 0.5 * x * y * y)
    return x * y


def _unpack_i32(v):
    va = lax.bitcast_convert_type(lax.shift_left(v, 16), jnp.float32)
    vb = lax.bitcast_convert_type(
        lax.bitwise_and(v, jnp.int32(-65536)), jnp.float32)
    return va, vb


def _sc_body(h_hbm, r_hbm, t_hbm, tm_hbm, st_hbm,
             ent_hbm, rel_hbm, modr_hbm, mode_hbm, time_hbm, step_hbm,
             out_hbm,
             idx_h, idx_r, idx_t, idx_tm, idx_st,
             step_tab, time_tab,
             hbuf, tbuf, hmbuf, rbuf, rmbuf,
             res_p, *sems):
    wid = lax.axis_index("s") * NC + lax.axis_index("c")
    lane = lax.iota(jnp.int32, 16)
    bfly = [lax.bitwise_and(lane + sh, 15)[:, None] for sh in (8, 4, 2, 1)]

    pltpu.sync_copy(h_hbm.at[wid], idx_h)
    pltpu.sync_copy(r_hbm.at[wid], idx_r)
    pltpu.sync_copy(t_hbm.at[wid], idx_t)
    pltpu.sync_copy(tm_hbm.at[wid], idx_tm.at[pl.ds(0, PW)])
    pltpu.sync_copy(st_hbm.at[wid], idx_st.at[pl.ds(0, PW)])
    pltpu.sync_copy(step_hbm, step_tab)
    pltpu.sync_copy(time_hbm, time_tab)

    for g in range(G):
        sl = pl.ds(g * 16, 16)
        step_tab[0, sl] = jnp.abs(step_tab[0, sl])

    def cum_body(i, carry):
        for g in range(G):
            sl = pl.ds(g * 16, 16)
            step_tab[i, sl] = jnp.abs(step_tab[i, sl]) + step_tab[i - 1, sl]
        return carry

    lax.fori_loop(1, NUM_MAXLEN, cum_body, 0)

    def chunk_copies(c, b):
        return [
            pltpu.make_async_copy(ent_hbm.at[idx_h.at[c]], hbuf.at[b], sems[b]),
            pltpu.make_async_copy(ent_hbm.at[idx_t.at[c]], tbuf.at[b], sems[b]),
            pltpu.make_async_copy(mode_hbm.at[idx_h.at[c]], hmbuf.at[b], sems[b]),
            pltpu.make_async_copy(rel_hbm.at[idx_r.at[c]], rbuf.at[b], sems[b]),
            pltpu.make_async_copy(modr_hbm.at[idx_r.at[c]], rmbuf.at[b],
                                  sems[b]),
        ]

    for b in range(NBUF):
        for cp in chunk_copies(b, b):
            cp.start()

    def compute_chunk(c, b):
        hb, tb, hmb, rb, rmb = (hbuf.at[b], tbuf.at[b], hmbuf.at[b],
                                rbuf.at[b], rmbuf.at[b])

        def grp_body(q, carry2):
            def elem_body(l, carry3):
                tp, tq = carry3
                j = q * 16 + l
                e = c * C + j
                tm_i = idx_tm[pl.ds(e, 16)][0]
                st_i = idx_st[pl.ds(e, 16)][0]
                accp = jnp.zeros((16,), jnp.float32)
                accm = jnp.zeros((16,), jnp.float32)
                for g in range(G):
                    sl = pl.ds(g * 16, 16)
                    h = hb[j, sl]
                    t = tb[j, sl]
                    tmv = time_tab[tm_i, sl]
                    stv = step_tab[st_i, sl]
                    w = (h - t) * (tmv + 1.0) + rb[j, sl]
                    accp = accp + _abs_sin_q(w)
                    dm = jnp.abs(hmb[j, sl]) * jnp.abs(rmb[j, sl]) - stv
                    accm = accm + dm * dm
                sel = lane == l
                tp = jnp.where(sel, _lane_sum(accp, bfly), tp)
                tq = jnp.where(sel, _lane_sum(accm, bfly), tq)
                return (tp, tq)

            z = jnp.zeros((16,), jnp.float32)
            tp, tq = lax.fori_loop(0, 16, elem_body, (z, z))
            base16 = c * C + q * 16
            res_p[pl.ds(base16, 16)] = tp + 0.5 * _sqrt(tq)
            return carry2

        lax.fori_loop(0, C // 16, grp_body, 0)

    def pipe_body(p, carry):
        for b in range(NBUF):
            c = NBUF * p + b
            for cp in chunk_copies(c, b):
                cp.wait()
            compute_chunk(c, b)
            nxt = jnp.minimum(c + NBUF, NCHUNK - 1)
            for cp in chunk_copies(nxt, b):
                cp.start()
        return carry

    lax.fori_loop(0, NCHUNK // NBUF, pipe_body, 0)
    for b in range(NBUF):
        for cp in chunk_copies(NCHUNK - 1, b):
            cp.wait()
    pltpu.sync_copy(res_p, out_hbm.at[wid])


@jax.jit
def _mpke_sc(h, r, t, tm, st, ent, rel, modr, mode, time_pk, stepw):
    mesh = plsc.VectorSubcoreMesh(core_axis_name="c", subcore_axis_name="s")
    f32 = jnp.float32
    i32 = jnp.int32
    run = pl.kernel(
        _sc_body,
        out_type=jax.ShapeDtypeStruct((NW, PW), f32),
        mesh=mesh,
        scratch_types=[
            pltpu.VMEM((NCHUNK, C), i32),
            pltpu.VMEM((NCHUNK, C), i32),
            pltpu.VMEM((NCHUNK, C), i32),
            pltpu.VMEM((PW + 16,), i32),
            pltpu.VMEM((PW + 16,), i32),
            pltpu.VMEM((NUM_MAXLEN, DIM), f32),
            pltpu.VMEM((NUM_TIME, DIM), f32),
            pltpu.VMEM((NBUF, C, DIM), f32),
            pltpu.VMEM((NBUF, C, DIM), f32),
            pltpu.VMEM((NBUF, C, DIM), f32),
            pltpu.VMEM((NBUF, C, DIM), f32),
            pltpu.VMEM((NBUF, C, DIM), f32),
            pltpu.VMEM((PW,), f32),
        ] + [pltpu.SemaphoreType.DMA] * NBUF,
    )
    return run(h, r, t, tm, st, ent, rel, modr, mode, time_pk, stepw)


def _pack_half(tab):
    rows = tab.shape[0]
    t = tab.reshape(rows, DIM // 32, 2, 16).transpose(0, 1, 3, 2)
    tb = t.astype(jnp.bfloat16)
    return lax.bitcast_convert_type(tb, jnp.int32).reshape(rows, DIM // 2)


def kernel(positiveBatchHead, positiveBatchRelation, positiveBatchTail,
           positiveBatchTime, positiveBatchStep,
           corruptedBatchHead, corruptedBatchRelation, corruptedBatchTail,
           corruptedBatchTime, corruptedBatchStep,
           entity_weight, relation_weight, mod_e_weight, mod_r_weight,
           time_weight, step_weight):
    def prep(a, b):
        return (jnp.concatenate([a, b]).astype(jnp.int32)
                .reshape(NW, NCHUNK, C))

    h = prep(positiveBatchHead, corruptedBatchHead)
    r = prep(positiveBatchRelation, corruptedBatchRelation)
    t = prep(positiveBatchTail, corruptedBatchTail)
    tm = prep(positiveBatchTime, corruptedBatchTime).reshape(NW, PW)
    st = prep(positiveBatchStep, corruptedBatchStep).reshape(NW, PW)
    out = _mpke_sc(h, r, t, tm, st, entity_weight, relation_weight,
                   mod_r_weight, mod_e_weight, time_weight, step_weight)
    loss = out.reshape(B2)
    return (loss[:BATCH], loss[BATCH:])
